# baseline (speedup 1.0000x reference)
import functools

import jax
import jax.numpy as jnp
from jax import lax
from jax.experimental import pallas as pl
from jax.experimental.pallas import tpu as pltpu
from jax.experimental.pallas import tpu_sc as plsc

B = 16384
F = 100
FP = 128
NFEAT = 1000000

NC = 2
NS = 16
NW = NC * NS
ROWS_W = B // NW
CHUNK = 128
NCHUNK = ROWS_W // CHUNK
CW = CHUNK * FP
GW = CHUNK * F
NJ = CHUNK // 16


def _build_sc_lr():
    mesh = plsc.VectorSubcoreMesh(core_axis_name="c", subcore_axis_name="s")

    @functools.partial(
        pl.kernel,
        mesh=mesh,
        compiler_params=pltpu.CompilerParams(needs_layout_passes=False),
        out_type=jax.ShapeDtypeStruct((B,), jnp.float32),
        scratch_types=[
            pltpu.VMEM((CW,), jnp.int32),
            pltpu.VMEM((CW,), jnp.int32),
            pltpu.VMEM((CW,), jnp.float32),
            pltpu.VMEM((CW,), jnp.float32),
            pltpu.VMEM((CW,), jnp.float32),
            pltpu.VMEM((CW,), jnp.float32),
            pltpu.VMEM((CHUNK,), jnp.float32),
            pltpu.SemaphoreType.DMA,
            pltpu.SemaphoreType.DMA,
        ],
    )
    def k(ids_hbm, vals_hbm, w_hbm, out_hbm,
          ids_v0, ids_v1, vals_v0, vals_v1, emb_v0, emb_v1, acc_v,
          sem0, sem1):
        wid = lax.axis_index("s") * NC + lax.axis_index("c")
        lane = lax.iota(jnp.int32, 16)
        ids_b = (ids_v0, ids_v1)
        vals_b = (vals_v0, vals_v1)
        emb_b = (emb_v0, emb_v1)
        sem_b = (sem0, sem1)

        def r0_of(c):
            return pl.multiple_of(wid * ROWS_W + c * CHUNK, 128)

        def fire_chunk(c):
            q = c % 2
            slab = pl.multiple_of(wid * (NCHUNK * CW) + c * CW, 128)
            pltpu.sync_copy(ids_hbm.at[pl.ds(slab, CW)], ids_b[q])

            def fire(j, carry):
                off = pl.multiple_of(j * FP, 128)
                pltpu.async_copy(
                    w_hbm.at[ids_b[q].at[pl.ds(off, F)]],
                    emb_b[q].at[pl.ds(off, F)],
                    sem_b[q],
                )
                return carry

            lax.fori_loop(0, CHUNK, fire, 0)
            pltpu.sync_copy(vals_hbm.at[pl.ds(slab, CW)], vals_b[q])

        fire_chunk(0)
        for c in range(NCHUNK):
            q = c % 2
            if c + 1 < NCHUNK:
                fire_chunk(c + 1)
            def drain(j, carry, q=q):
                pltpu.make_async_copy(
                    out_hbm.at[pl.ds(0, CHUNK)], acc_v, sem_b[q]
                ).wait()
                return carry

            lax.fori_loop(0, F, drain, 0)

            pjs = tuple(lane * FP + j * 16 * FP for j in range(NJ))

            def f_body(f, accs, q=q, pjs=pjs):
                out = []
                for j in range(NJ):
                    p = pjs[j] + f
                    e = plsc.load_gather(emb_b[q], [p])
                    v = plsc.load_gather(vals_b[q], [p])
                    out.append(accs[j] + e * v)
                return tuple(out)

            accs = lax.fori_loop(
                0, F, f_body, (jnp.zeros((16,), jnp.float32),) * NJ
            )
            for j in range(NJ):
                acc_v[pl.ds(j * 16, 16)] = accs[j]
            pltpu.sync_copy(acc_v, out_hbm.at[pl.ds(r0_of(c), CHUNK)])

    return k


_SC_LR = _build_sc_lr()


def kernel(ids, vals, weight, bias):
    ids1 = jnp.pad(ids.astype(jnp.int32), ((0, 0), (0, FP - F))).reshape(B * FP)
    vals1 = jnp.pad(vals, ((0, 0), (0, FP - F))).reshape(B * FP)
    w1 = weight.reshape(NFEAT)
    y = _SC_LR(ids1, vals1, w1)
    return y + bias


# speedup vs baseline: 1.0782x; 1.0782x over previous
import functools

import jax
import jax.numpy as jnp
from jax import lax
from jax.experimental import pallas as pl
from jax.experimental.pallas import tpu as pltpu
from jax.experimental.pallas import tpu_sc as plsc

B = 16384
F = 100
FP = 128
NFEAT = 1000000

NC = 2
NS = 16
NW = NC * NS
ROWS_W = B // NW
CHUNK = 128
NCHUNK = ROWS_W // CHUNK
CW = CHUNK * FP
GW = CHUNK * F
DW = GW + 16
NROW = GW // 128
NJ = CHUNK // 16
NK = 112 // 16


def _build_sc_lr():
    mesh = plsc.VectorSubcoreMesh(core_axis_name="c", subcore_axis_name="s")

    @functools.partial(
        pl.kernel,
        mesh=mesh,
        compiler_params=pltpu.CompilerParams(needs_layout_passes=False),
        out_type=jax.ShapeDtypeStruct((B,), jnp.float32),
        scratch_types=[
            pltpu.VMEM((CW,), jnp.int32),
            pltpu.VMEM((DW,), jnp.int32),
            pltpu.VMEM((DW,), jnp.int32),
            pltpu.VMEM((CW,), jnp.float32),
            pltpu.VMEM((CW,), jnp.float32),
            pltpu.VMEM((DW,), jnp.float32),
            pltpu.VMEM((DW,), jnp.float32),
            pltpu.VMEM((CHUNK,), jnp.float32),
            pltpu.SemaphoreType.DMA,
            pltpu.SemaphoreType.DMA,
        ],
    )
    def k(ids_hbm, vals_hbm, w_hbm, out_hbm,
          ids_p, idsd_v0, idsd_v1, vals_v0, vals_v1, emb_v0, emb_v1, acc_v,
          sem0, sem1):
        wid = lax.axis_index("s") * NC + lax.axis_index("c")
        lane = lax.iota(jnp.int32, 16)
        idsd_b = (idsd_v0, idsd_v1)
        vals_b = (vals_v0, vals_v1)
        emb_b = (emb_v0, emb_v1)
        sem_b = (sem0, sem1)

        def r0_of(c):
            return pl.multiple_of(wid * ROWS_W + c * CHUNK, 128)

        def fire_chunk(c):
            q = c % 2
            slab = pl.multiple_of(wid * (NCHUNK * CW) + c * CW, 128)
            pltpu.sync_copy(ids_hbm.at[pl.ds(slab, CW)], ids_p)

            def compact(r, carry, q=q):
                src = pl.multiple_of(r * FP, 128)
                dst = r * F + lane
                for kk in range(NK):
                    v = ids_p[pl.ds(src + kk * 16, 16)]
                    plsc.store_scatter(idsd_b[q], [dst + kk * 16], v)
                return carry

            lax.fori_loop(0, CHUNK, compact, 0)

            def fire(j, carry, q=q):
                off = pl.multiple_of(j * 128, 128)
                pltpu.async_copy(
                    w_hbm.at[idsd_b[q].at[pl.ds(off, 128)]],
                    emb_b[q].at[pl.ds(off, 128)],
                    sem_b[q],
                )
                return carry

            lax.fori_loop(0, NROW, fire, 0)
            pltpu.sync_copy(vals_hbm.at[pl.ds(slab, CW)], vals_b[q])

        fire_chunk(0)
        for c in range(NCHUNK):
            q = c % 2
            if c + 1 < NCHUNK:
                fire_chunk(c + 1)
            def drain(j, carry, q=q):
                pltpu.make_async_copy(
                    out_hbm.at[pl.ds(0, CHUNK)], acc_v, sem_b[q]
                ).wait()
                return carry

            lax.fori_loop(0, F, drain, 0)

            pj100 = tuple(lane * F + j * 16 * F for j in range(NJ))
            pj128 = tuple(lane * FP + j * 16 * FP for j in range(NJ))

            def f_body(f, accs, q=q, pj100=pj100, pj128=pj128):
                out = []
                for j in range(NJ):
                    e = plsc.load_gather(emb_b[q], [pj100[j] + f])
                    v = plsc.load_gather(vals_b[q], [pj128[j] + f])
                    out.append(accs[j] + e * v)
                return tuple(out)

            accs = lax.fori_loop(
                0, F, f_body, (jnp.zeros((16,), jnp.float32),) * NJ
            )
            for j in range(NJ):
                acc_v[pl.ds(j * 16, 16)] = accs[j]
            pltpu.sync_copy(acc_v, out_hbm.at[pl.ds(r0_of(c), CHUNK)])

    return k


_SC_LR = _build_sc_lr()


def kernel(ids, vals, weight, bias):
    ids1 = jnp.pad(ids.astype(jnp.int32), ((0, 0), (0, FP - F))).reshape(B * FP)
    vals1 = jnp.pad(vals, ((0, 0), (0, FP - F))).reshape(B * FP)
    w1 = weight.reshape(NFEAT)
    y = _SC_LR(ids1, vals1, w1)
    return y + bias


# speedup vs baseline: 1.4829x; 1.3754x over previous
import functools

import jax
import jax.numpy as jnp
from jax import lax
from jax.experimental import pallas as pl
from jax.experimental.pallas import tpu as pltpu
from jax.experimental.pallas import tpu_sc as plsc

B = 16384
F = 100
NFEAT = 1000000

NC = 2
NS = 16
NW = NC * NS
ROWS_W = B // NW
CHUNK = 64
NCHUNK = ROWS_W // CHUNK
CW = CHUNK * F
NROW = CW // 128
NJ = CHUNK // 16


def _build_sc_lr():
    mesh = plsc.VectorSubcoreMesh(core_axis_name="c", subcore_axis_name="s")

    @functools.partial(
        pl.kernel,
        mesh=mesh,
        compiler_params=pltpu.CompilerParams(needs_layout_passes=False),
        out_type=jax.ShapeDtypeStruct((B,), jnp.float32),
        scratch_types=[
            pltpu.VMEM((CW,), jnp.int32),
            pltpu.VMEM((CW,), jnp.int32),
            pltpu.VMEM((CW,), jnp.float32),
            pltpu.VMEM((CW,), jnp.float32),
            pltpu.VMEM((CW,), jnp.float32),
            pltpu.VMEM((CW,), jnp.float32),
            pltpu.VMEM((CHUNK,), jnp.float32),
            pltpu.VMEM((8192,), jnp.float32),
            pltpu.VMEM_SHARED((NFEAT,), jnp.float32),
            pltpu.SemaphoreType.DMA,
            pltpu.SemaphoreType.DMA,
        ],
    )
    def k(ids_hbm, vals_hbm, w_hbm, out_hbm,
          ids_v0, ids_v1, vals_v0, vals_v1, emb_v0, emb_v1, acc_v, stg_v,
          w_sh, sem0, sem1):
        wid = lax.axis_index("s") * NC + lax.axis_index("c")
        sid = lax.axis_index("s")
        lane = lax.iota(jnp.int32, 16)
        ids_b = (ids_v0, ids_v1)
        vals_b = (vals_v0, vals_v1)
        emb_b = (emb_v0, emb_v1)
        sem_b = (sem0, sem1)

        PIECE = 8192
        NPIECE = NFEAT // PIECE
        TAIL = NFEAT - NPIECE * PIECE

        def stage(kk, carry):
            p = sid + NS * kk

            @pl.when(p < NPIECE)
            def _copy_piece():
                off = pl.multiple_of(p * PIECE, 128)
                pltpu.sync_copy(w_hbm.at[pl.ds(off, PIECE)], stg_v)
                pltpu.sync_copy(stg_v, w_sh.at[pl.ds(off, PIECE)])

            return carry

        lax.fori_loop(0, (NPIECE + NS - 1) // NS, stage, 0)

        @pl.when(sid == 0)
        def _stage_tail():
            toff = pl.multiple_of(NPIECE * PIECE, 128)
            pltpu.sync_copy(
                w_hbm.at[pl.ds(toff, TAIL)], stg_v.at[pl.ds(0, TAIL)]
            )
            pltpu.sync_copy(
                stg_v.at[pl.ds(0, TAIL)], w_sh.at[pl.ds(toff, TAIL)]
            )

        plsc.subcore_barrier()

        def slab_of(c):
            return pl.multiple_of(wid * (NCHUNK * CW) + c * CW, 128)

        def fire_chunk(c):
            q = c % 2
            slab = slab_of(c)
            pltpu.sync_copy(ids_hbm.at[pl.ds(slab, CW)], ids_b[q])

            def fire(j, carry):
                off = pl.multiple_of(j * 128, 128)
                pltpu.async_copy(
                    w_sh.at[ids_b[q].at[pl.ds(off, 128)]],
                    emb_b[q].at[pl.ds(off, 128)],
                    sem_b[q],
                )
                return carry

            lax.fori_loop(0, NROW, fire, 0)
            pltpu.sync_copy(vals_hbm.at[pl.ds(slab, CW)], vals_b[q])

        fire_chunk(0)
        for c in range(NCHUNK):
            q = c % 2
            if c + 1 < NCHUNK:
                fire_chunk(c + 1)
            pltpu.make_async_copy(
                vals_hbm.at[pl.ds(slab_of(c), CW)], emb_b[q], sem_b[q]
            ).wait()

            pjs = tuple(lane * F + j * 16 * F for j in range(NJ))

            def f_body(f, accs, q=q, pjs=pjs):
                out = []
                for j in range(NJ):
                    p = pjs[j] + f
                    e = plsc.load_gather(emb_b[q], [p])
                    v = plsc.load_gather(vals_b[q], [p])
                    out.append(accs[j] + e * v)
                return tuple(out)

            accs = lax.fori_loop(
                0, F, f_body, (jnp.zeros((16,), jnp.float32),) * NJ
            )
            for j in range(NJ):
                acc_v[pl.ds(j * 16, 16)] = accs[j]
            r0 = pl.multiple_of(wid * ROWS_W + c * CHUNK, CHUNK)
            pltpu.sync_copy(acc_v, out_hbm.at[pl.ds(r0, CHUNK)])

    return k


_SC_LR = _build_sc_lr()


def kernel(ids, vals, weight, bias):
    ids1 = ids.astype(jnp.int32).reshape(B * F)
    vals1 = vals.reshape(B * F)
    w1 = weight.reshape(NFEAT)
    y = _SC_LR(ids1, vals1, w1)
    return y + bias
